# baseline (speedup 1.0000x reference)
import functools

import jax
import jax.numpy as jnp
from jax import lax
from jax.experimental import pallas as pl
from jax.experimental.pallas import tpu as pltpu
from jax.experimental.pallas import tpu_sc as plsc

N = 10000
NP = 10240
E = 160000
B = 128
DIN = 1280
D = 256

NC = 2
NS = 16
LANES = 16

NPT = NP // (NC * NS)
DEG_CHUNK = 16000


def _rsqrt_newton(d):
    i = plsc.bitcast(d, jnp.int32)
    i = jnp.int32(0x5F3759DF) - (i >> 1)
    y = plsc.bitcast(i, jnp.float32)
    for _ in range(3):
        y = y * (1.5 - 0.5 * d * y * y)
    return y


def _deg_body(dst_hbm, dinv_hbm, dstv, deg, dinvv):
    c = lax.axis_index("c")
    s = lax.axis_index("s")
    base = (c * NS + s) * NPT
    zero16 = jnp.zeros((LANES,), jnp.float32)
    ones16 = jnp.ones((LANES,), jnp.float32)
    for i in range(NPT // LANES):
        deg[pl.ds(i * LANES, LANES)] = zero16

    for ch in range(E // DEG_CHUNK):
        pltpu.sync_copy(dst_hbm.at[pl.ds(ch * DEG_CHUNK, DEG_CHUNK)], dstv)

        def body(j, carry):
            v = dstv[pl.ds(j * LANES, LANES)]
            dloc = v - base
            m = (dloc >= 0) & (dloc < NPT)
            dloc = jnp.where(m, dloc, 0)
            plsc.addupdate_scatter(deg, [dloc], ones16, mask=m)
            return carry

        lax.fori_loop(0, DEG_CHUNK // LANES, body, 0)

    iota = lax.broadcasted_iota(jnp.int32, (LANES,), 0)
    for i in range(NPT // LANES):
        dg = deg[pl.ds(i * LANES, LANES)] + 1.0
        y = _rsqrt_newton(dg)
        nid = base + i * LANES + iota
        dinvv[pl.ds(i * LANES, LANES)] = jnp.where(nid < N, y, 0.0)
    pltpu.sync_copy(dinvv, dinv_hbm.at[pl.ds(base, NPT)])


def _sc_degree(dst):
    mesh = plsc.VectorSubcoreMesh(core_axis_name="c", subcore_axis_name="s",
                                  num_cores=NC, num_subcores=NS)
    f = pl.kernel(
        _deg_body,
        out_type=jax.ShapeDtypeStruct((NP,), jnp.float32),
        mesh=mesh,
        compiler_params=pltpu.CompilerParams(needs_layout_passes=False),
        scratch_types=[
            pltpu.VMEM((DEG_CHUNK,), jnp.int32),
            pltpu.VMEM((NPT,), jnp.float32),
            pltpu.VMEM((NPT,), jnp.float32),
        ],
    )
    return f(dst)


TPW = NP // (NC * NS)
TRASH_L = TPW
ACC_R = 336
KB = 96
CH = 2000
NCH = E // CH
STG = CH + KB + LANES


def _scatter_body(hws_hbm, src_hbm, dst_hbm, zer_hbm, x2_hbm,
                  srcv, dstv, cidx, cdloc, rows, acc, sem):
    c = lax.axis_index("c")
    s = lax.axis_index("s")
    lo = (c * NS + s) * TPW
    pltpu.sync_copy(zer_hbm, acc)

    trash16 = jnp.full((LANES,), TRASH_L, jnp.int32)
    pad16 = jnp.zeros((LANES,), jnp.int32)
    iota16 = lax.broadcasted_iota(jnp.int32, (LANES,), 0)

    def chunk_body(ch, carry):
        pltpu.sync_copy(src_hbm.at[pl.ds(ch * CH, CH)], srcv)
        pltpu.sync_copy(dst_hbm.at[pl.ds(ch * CH, CH)], dstv)

        def comp(j, cnt):
            sv = srcv[pl.ds(j * LANES, LANES)]
            dv = dstv[pl.ds(j * LANES, LANES)]
            dl = dv - lo
            m = (dl >= 0) & (dl < TPW)
            plsc.store_compressed(cidx.at[pl.ds(cnt, LANES)], sv, mask=m)
            plsc.store_compressed(cdloc.at[pl.ds(cnt, LANES)],
                                  jnp.where(m, dl, 0), mask=m)
            pc = plsc.all_reduce_population_count(m)
            return cnt + jnp.squeeze(lax.slice(pc, (0,), (1,)))

        cnt = lax.fori_loop(0, CH // LANES, comp, jnp.int32(0))
        for j in range(KB // LANES):
            cidx[pl.ds(cnt + j * LANES, LANES)] = pad16
            cdloc[pl.ds(cnt + j * LANES, LANES)] = trash16

        def bat(b, carry2):
            pltpu.async_copy(hws_hbm.at[cidx.at[pl.ds(b * KB, KB)]],
                             rows, sem).wait()
            for g in range(KB // LANES):
                dl16 = cdloc[pl.ds(b * KB + g * LANES, LANES)]
                e16 = iota16 + g * LANES

                def _feat(j):
                    for jj in range(LANES):
                        fv = jnp.full((LANES,), 0, jnp.int32) + (j + jj)
                        v = plsc.load_gather(rows, [e16, fv])
                        plsc.addupdate_scatter(acc, [dl16, fv], v)

                plsc.parallel_loop(0, D, LANES, unroll=2)(_feat)

            return carry2

        lax.fori_loop(0, (cnt + KB - 1) // KB, bat, 0)
        return carry

    lax.fori_loop(0, NCH, chunk_body, 0)
    pltpu.sync_copy(acc.at[pl.ds(0, TPW)], x2_hbm.at[pl.ds(lo, TPW)])


def _sc_scatter(hws, src, dst, zer):
    mesh = plsc.VectorSubcoreMesh(core_axis_name="c", subcore_axis_name="s",
                                  num_cores=NC, num_subcores=NS)
    f = pl.kernel(
        _scatter_body,
        out_type=jax.ShapeDtypeStruct((NP, D), jnp.float32),
        mesh=mesh,
        compiler_params=pltpu.CompilerParams(needs_layout_passes=False,
                                             disable_bounds_checks=True),
        scratch_types=[
            pltpu.VMEM((CH,), jnp.int32),
            pltpu.VMEM((CH,), jnp.int32),
            pltpu.VMEM((STG,), jnp.int32),
            pltpu.VMEM((STG,), jnp.int32),
            pltpu.VMEM((KB, D), jnp.float32),
            pltpu.VMEM((ACC_R, D), jnp.float32),
            pltpu.SemaphoreType.DMA,
        ],
    )
    return f(hws, src, dst, zer)


BM = 512


def _mm_body(x_ref, wt_ref, bt_ref, wg_ref, dinv_ref, hw_ref, hws_ref):
    h = jnp.dot(x_ref[...], wt_ref[...], preferred_element_type=jnp.float32)
    h = jnp.maximum(h + bt_ref[...], 0.0)
    hw = jnp.dot(h, wg_ref[...], preferred_element_type=jnp.float32)
    hw_ref[...] = hw
    hws_ref[...] = hw * dinv_ref[...]


def _tc_matmul(xp, Wt, bt2, Wg, dinv2):
    return pl.pallas_call(
        _mm_body,
        grid=(NP // BM,),
        in_specs=[
            pl.BlockSpec((BM, DIN), lambda i: (i, 0)),
            pl.BlockSpec((DIN, D), lambda i: (0, 0)),
            pl.BlockSpec((1, D), lambda i: (0, 0)),
            pl.BlockSpec((D, D), lambda i: (0, 0)),
            pl.BlockSpec((BM, 1), lambda i: (i, 0)),
        ],
        out_specs=[
            pl.BlockSpec((BM, D), lambda i: (i, 0)),
            pl.BlockSpec((BM, D), lambda i: (i, 0)),
        ],
        out_shape=[
            jax.ShapeDtypeStruct((NP, D), jnp.float32),
            jax.ShapeDtypeStruct((NP, D), jnp.float32),
        ],
    )(xp, Wt, bt2, Wg, dinv2)


BN = 1024


def _final_body(x2r_ref, hw_ref, dinv_ref, batch_ref, root_ref, bg_ref,
                wv1_ref, bv1_ref, wo1_ref, bo1_ref,
                wv2_ref, bv2_ref, wo2_ref, bo2_ref,
                out_ref, sacc, racc, cacc):
    i = pl.program_id(0)

    @pl.when(i == 0)
    def _init():
        sacc[...] = jnp.zeros_like(sacc)
        racc[...] = jnp.zeros_like(racc)
        cacc[...] = jnp.zeros_like(cacc)

    dv = dinv_ref[...]
    x2 = x2r_ref[...] * dv + hw_ref[...] * (dv * dv) + bg_ref[...]
    xr = jnp.maximum(x2, 0.0)

    bt = batch_ref[...]
    gid = lax.broadcasted_iota(jnp.int32, (B, BN), 0)
    sel = (bt == gid).astype(jnp.float32)
    sacc[...] += jnp.dot(sel, xr, preferred_element_type=jnp.float32)
    cacc[...] += jnp.sum(sel, axis=1, keepdims=True)

    nid = lax.broadcasted_iota(jnp.int32, (B, BN), 1) + i * BN
    rsel = (root_ref[...] == nid).astype(jnp.float32)
    racc[...] += jnp.dot(rsel, x2, preferred_element_type=jnp.float32)

    @pl.when(i == NP // BN - 1)
    def _fin():
        cnt = cacc[...]
        mean = sacc[...] / jnp.maximum(cnt, 1.0)
        o1 = jnp.dot(mean, wv1_ref[...], preferred_element_type=jnp.float32)
        o1 = jnp.dot(o1 + bv1_ref[...], wo1_ref[...],
                     preferred_element_type=jnp.float32) + bo1_ref[...]
        o2 = jnp.dot(racc[...], wv2_ref[...], preferred_element_type=jnp.float32)
        o2 = jnp.dot(o2 + bv2_ref[...], wo2_ref[...],
                     preferred_element_type=jnp.float32) + bo2_ref[...]
        out_ref[...] = jnp.where(cnt > 0.0,
                                 jnp.concatenate([o1, o2], axis=1), 0.0)


def _tc_final(x2raw, hw, dinv2, batchT, root2, bg2,
              Wv1, bv12, Wo1, bo12, Wv2, bv22, Wo2, bo22):
    full = lambda shape: pl.BlockSpec(shape, lambda i: (0, 0))
    return pl.pallas_call(
        _final_body,
        grid=(NP // BN,),
        in_specs=[
            pl.BlockSpec((BN, D), lambda i: (i, 0)),
            pl.BlockSpec((BN, D), lambda i: (i, 0)),
            pl.BlockSpec((BN, 1), lambda i: (i, 0)),
            pl.BlockSpec((1, BN), lambda i: (0, i)),
            full((B, 1)), full((1, D)),
            full((D, D)), full((1, D)), full((D, D)), full((1, D)),
            full((D, D)), full((1, D)), full((D, D)), full((1, D)),
        ],
        out_specs=pl.BlockSpec((B, 2 * D), lambda i: (0, 0)),
        out_shape=jax.ShapeDtypeStruct((B, 2 * D), jnp.float32),
        scratch_shapes=[
            pltpu.VMEM((B, D), jnp.float32),
            pltpu.VMEM((B, D), jnp.float32),
            pltpu.VMEM((B, 1), jnp.float32),
        ],
    )(x2raw, hw, dinv2, batchT, root2, bg2,
      Wv1, bv12, Wo1, bo12, Wv2, bv22, Wo2, bo22)


def kernel(x, edge_index, rootindex, batch, W_text, b_text, W_gcn, b_gcn,
           Wq1, bq1, Wk1, bk1, Wv1, bv1, Wq2, bq2, Wk2, bk2, Wv2, bv2,
           Wo1, bo1, Wo2, bo2):
    xp = jnp.pad(x, ((0, NP - N), (0, 0)))
    src = edge_index[0]
    dst = edge_index[1]

    dinv = _sc_degree(dst)
    dinv2 = dinv.reshape(NP, 1)
    hw, hws = _tc_matmul(xp, W_text, b_text.reshape(1, D), W_gcn, dinv2)

    zer = jnp.zeros((ACC_R, D), jnp.float32)
    x2raw = _sc_scatter(hws, src, dst, zer)

    batchT = jnp.pad(batch, (0, NP - N), constant_values=B).reshape(1, NP)
    return _tc_final(x2raw, hw, dinv2, batchT,
                     rootindex.reshape(B, 1), b_gcn.reshape(1, D),
                     Wv1, bv1.reshape(1, D), Wo1, bo1.reshape(1, D),
                     Wv2, bv2.reshape(1, D), Wo2, bo2.reshape(1, D))


# speedup vs baseline: 1.0006x; 1.0006x over previous
import functools

import jax
import jax.numpy as jnp
from jax import lax
from jax.experimental import pallas as pl
from jax.experimental.pallas import tpu as pltpu
from jax.experimental.pallas import tpu_sc as plsc

N = 10000
NP = 10240
E = 160000
B = 128
DIN = 1280
D = 256

NC = 2
NS = 16
LANES = 16

NPT = NP // (NC * NS)
DEG_CHUNK = 16000


def _rsqrt_newton(d):
    i = plsc.bitcast(d, jnp.int32)
    i = jnp.int32(0x5F3759DF) - (i >> 1)
    y = plsc.bitcast(i, jnp.float32)
    for _ in range(3):
        y = y * (1.5 - 0.5 * d * y * y)
    return y


def _deg_body(dst_hbm, dinv_hbm, dstv, deg, dinvv):
    c = lax.axis_index("c")
    s = lax.axis_index("s")
    base = (c * NS + s) * NPT
    zero16 = jnp.zeros((LANES,), jnp.float32)
    ones16 = jnp.ones((LANES,), jnp.float32)
    for i in range(NPT // LANES):
        deg[pl.ds(i * LANES, LANES)] = zero16

    for ch in range(E // DEG_CHUNK):
        pltpu.sync_copy(dst_hbm.at[pl.ds(ch * DEG_CHUNK, DEG_CHUNK)], dstv)

        def body(j, carry):
            v = dstv[pl.ds(j * LANES, LANES)]
            dloc = v - base
            m = (dloc >= 0) & (dloc < NPT)
            dloc = jnp.where(m, dloc, 0)
            plsc.addupdate_scatter(deg, [dloc], ones16, mask=m)
            return carry

        lax.fori_loop(0, DEG_CHUNK // LANES, body, 0)

    iota = lax.broadcasted_iota(jnp.int32, (LANES,), 0)
    for i in range(NPT // LANES):
        dg = deg[pl.ds(i * LANES, LANES)] + 1.0
        y = _rsqrt_newton(dg)
        nid = base + i * LANES + iota
        dinvv[pl.ds(i * LANES, LANES)] = jnp.where(nid < N, y, 0.0)
    pltpu.sync_copy(dinvv, dinv_hbm.at[pl.ds(base, NPT)])


def _sc_degree(dst):
    mesh = plsc.VectorSubcoreMesh(core_axis_name="c", subcore_axis_name="s",
                                  num_cores=NC, num_subcores=NS)
    f = pl.kernel(
        _deg_body,
        out_type=jax.ShapeDtypeStruct((NP,), jnp.float32),
        mesh=mesh,
        compiler_params=pltpu.CompilerParams(needs_layout_passes=False),
        scratch_types=[
            pltpu.VMEM((DEG_CHUNK,), jnp.int32),
            pltpu.VMEM((NPT,), jnp.float32),
            pltpu.VMEM((NPT,), jnp.float32),
        ],
    )
    return f(dst)


TPW = NP // (NC * NS)
TRASH_L = TPW
ACC_R = 336
KB = 96
CH = 2000
NCH = E // CH
STG = CH + KB + LANES


def _scatter_body(hws_hbm, src_hbm, dst_hbm, zer_hbm, x2_hbm,
                  srcv, dstv, cidx, cdloc, rows, acc, sem):
    c = lax.axis_index("c")
    s = lax.axis_index("s")
    lo = (c * NS + s) * TPW
    pltpu.sync_copy(zer_hbm, acc)

    trash16 = jnp.full((LANES,), TRASH_L, jnp.int32)
    pad16 = jnp.zeros((LANES,), jnp.int32)
    iota16 = lax.broadcasted_iota(jnp.int32, (LANES,), 0)

    def chunk_body(ch, carry):
        with jax.named_scope("edges_dma"):
            pltpu.sync_copy(src_hbm.at[pl.ds(ch * CH, CH)], srcv)
            pltpu.sync_copy(dst_hbm.at[pl.ds(ch * CH, CH)], dstv)

        def comp(j, cnt):
            sv = srcv[pl.ds(j * LANES, LANES)]
            dv = dstv[pl.ds(j * LANES, LANES)]
            dl = dv - lo
            m = (dl >= 0) & (dl < TPW)
            plsc.store_compressed(cidx.at[pl.ds(cnt, LANES)], sv, mask=m)
            plsc.store_compressed(cdloc.at[pl.ds(cnt, LANES)],
                                  jnp.where(m, dl, 0), mask=m)
            pc = plsc.all_reduce_population_count(m)
            return cnt + jnp.squeeze(lax.slice(pc, (0,), (1,)))

        with jax.named_scope("compact"):
            cnt = lax.fori_loop(0, CH // LANES, comp, jnp.int32(0))
            for j in range(KB // LANES):
                cidx[pl.ds(cnt + j * LANES, LANES)] = pad16
                cdloc[pl.ds(cnt + j * LANES, LANES)] = trash16

        def bat(b, carry2):
            with jax.named_scope("gather"):
                pltpu.async_copy(hws_hbm.at[cidx.at[pl.ds(b * KB, KB)]],
                                 rows, sem).wait()
            with jax.named_scope("accum"):
                for g in range(KB // LANES):
                    dl16 = cdloc[pl.ds(b * KB + g * LANES, LANES)]
                    e16 = iota16 + g * LANES

                    def _feat(j):
                        for jj in range(LANES):
                            fv = jnp.full((LANES,), 0, jnp.int32) + (j + jj)
                            v = plsc.load_gather(rows, [e16, fv])
                            plsc.addupdate_scatter(acc, [dl16, fv], v)

                    plsc.parallel_loop(0, D, LANES, unroll=2)(_feat)

            return carry2

        lax.fori_loop(0, (cnt + KB - 1) // KB, bat, 0)
        return carry

    lax.fori_loop(0, NCH, chunk_body, 0)
    pltpu.sync_copy(acc.at[pl.ds(0, TPW)], x2_hbm.at[pl.ds(lo, TPW)])


def _sc_scatter(hws, src, dst, zer):
    mesh = plsc.VectorSubcoreMesh(core_axis_name="c", subcore_axis_name="s",
                                  num_cores=NC, num_subcores=NS)
    f = pl.kernel(
        _scatter_body,
        out_type=jax.ShapeDtypeStruct((NP, D), jnp.float32),
        mesh=mesh,
        compiler_params=pltpu.CompilerParams(needs_layout_passes=False,
                                             disable_bounds_checks=True),
        scratch_types=[
            pltpu.VMEM((CH,), jnp.int32),
            pltpu.VMEM((CH,), jnp.int32),
            pltpu.VMEM((STG,), jnp.int32),
            pltpu.VMEM((STG,), jnp.int32),
            pltpu.VMEM((KB, D), jnp.float32),
            pltpu.VMEM((ACC_R, D), jnp.float32),
            pltpu.SemaphoreType.DMA,
        ],
    )
    return f(hws, src, dst, zer)


BM = 512


def _mm_body(x_ref, wt_ref, bt_ref, wg_ref, dinv_ref, hw_ref, hws_ref):
    h = jnp.dot(x_ref[...], wt_ref[...], preferred_element_type=jnp.float32)
    h = jnp.maximum(h + bt_ref[...], 0.0)
    hw = jnp.dot(h, wg_ref[...], preferred_element_type=jnp.float32)
    hw_ref[...] = hw
    hws_ref[...] = hw * dinv_ref[...]


def _tc_matmul(xp, Wt, bt2, Wg, dinv2):
    return pl.pallas_call(
        _mm_body,
        grid=(NP // BM,),
        in_specs=[
            pl.BlockSpec((BM, DIN), lambda i: (i, 0)),
            pl.BlockSpec((DIN, D), lambda i: (0, 0)),
            pl.BlockSpec((1, D), lambda i: (0, 0)),
            pl.BlockSpec((D, D), lambda i: (0, 0)),
            pl.BlockSpec((BM, 1), lambda i: (i, 0)),
        ],
        out_specs=[
            pl.BlockSpec((BM, D), lambda i: (i, 0)),
            pl.BlockSpec((BM, D), lambda i: (i, 0)),
        ],
        out_shape=[
            jax.ShapeDtypeStruct((NP, D), jnp.float32),
            jax.ShapeDtypeStruct((NP, D), jnp.float32),
        ],
    )(xp, Wt, bt2, Wg, dinv2)


BN = 1024


def _final_body(x2r_ref, hw_ref, dinv_ref, batch_ref, root_ref, bg_ref,
                wv1_ref, bv1_ref, wo1_ref, bo1_ref,
                wv2_ref, bv2_ref, wo2_ref, bo2_ref,
                out_ref, sacc, racc, cacc):
    i = pl.program_id(0)

    @pl.when(i == 0)
    def _init():
        sacc[...] = jnp.zeros_like(sacc)
        racc[...] = jnp.zeros_like(racc)
        cacc[...] = jnp.zeros_like(cacc)

    dv = dinv_ref[...]
    x2 = x2r_ref[...] * dv + hw_ref[...] * (dv * dv) + bg_ref[...]
    xr = jnp.maximum(x2, 0.0)

    bt = batch_ref[...]
    gid = lax.broadcasted_iota(jnp.int32, (B, BN), 0)
    sel = (bt == gid).astype(jnp.float32)
    sacc[...] += jnp.dot(sel, xr, preferred_element_type=jnp.float32)
    cacc[...] += jnp.sum(sel, axis=1, keepdims=True)

    nid = lax.broadcasted_iota(jnp.int32, (B, BN), 1) + i * BN
    rsel = (root_ref[...] == nid).astype(jnp.float32)
    racc[...] += jnp.dot(rsel, x2, preferred_element_type=jnp.float32)

    @pl.when(i == NP // BN - 1)
    def _fin():
        cnt = cacc[...]
        mean = sacc[...] / jnp.maximum(cnt, 1.0)
        o1 = jnp.dot(mean, wv1_ref[...], preferred_element_type=jnp.float32)
        o1 = jnp.dot(o1 + bv1_ref[...], wo1_ref[...],
                     preferred_element_type=jnp.float32) + bo1_ref[...]
        o2 = jnp.dot(racc[...], wv2_ref[...], preferred_element_type=jnp.float32)
        o2 = jnp.dot(o2 + bv2_ref[...], wo2_ref[...],
                     preferred_element_type=jnp.float32) + bo2_ref[...]
        out_ref[...] = jnp.where(cnt > 0.0,
                                 jnp.concatenate([o1, o2], axis=1), 0.0)


def _tc_final(x2raw, hw, dinv2, batchT, root2, bg2,
              Wv1, bv12, Wo1, bo12, Wv2, bv22, Wo2, bo22):
    full = lambda shape: pl.BlockSpec(shape, lambda i: (0, 0))
    return pl.pallas_call(
        _final_body,
        grid=(NP // BN,),
        in_specs=[
            pl.BlockSpec((BN, D), lambda i: (i, 0)),
            pl.BlockSpec((BN, D), lambda i: (i, 0)),
            pl.BlockSpec((BN, 1), lambda i: (i, 0)),
            pl.BlockSpec((1, BN), lambda i: (0, i)),
            full((B, 1)), full((1, D)),
            full((D, D)), full((1, D)), full((D, D)), full((1, D)),
            full((D, D)), full((1, D)), full((D, D)), full((1, D)),
        ],
        out_specs=pl.BlockSpec((B, 2 * D), lambda i: (0, 0)),
        out_shape=jax.ShapeDtypeStruct((B, 2 * D), jnp.float32),
        scratch_shapes=[
            pltpu.VMEM((B, D), jnp.float32),
            pltpu.VMEM((B, D), jnp.float32),
            pltpu.VMEM((B, 1), jnp.float32),
        ],
    )(x2raw, hw, dinv2, batchT, root2, bg2,
      Wv1, bv12, Wo1, bo12, Wv2, bv22, Wo2, bo22)


def kernel(x, edge_index, rootindex, batch, W_text, b_text, W_gcn, b_gcn,
           Wq1, bq1, Wk1, bk1, Wv1, bv1, Wq2, bq2, Wk2, bk2, Wv2, bv2,
           Wo1, bo1, Wo2, bo2):
    xp = jnp.pad(x, ((0, NP - N), (0, 0)))
    src = edge_index[0]
    dst = edge_index[1]

    dinv = _sc_degree(dst)
    dinv2 = dinv.reshape(NP, 1)
    hw, hws = _tc_matmul(xp, W_text, b_text.reshape(1, D), W_gcn, dinv2)

    zer = jnp.zeros((ACC_R, D), jnp.float32)
    x2raw = _sc_scatter(hws, src, dst, zer)

    batchT = jnp.pad(batch, (0, NP - N), constant_values=B).reshape(1, NP)
    return _tc_final(x2raw, hw, dinv2, batchT,
                     rootindex.reshape(B, 1), b_gcn.reshape(1, D),
                     Wv1, bv1.reshape(1, D), Wo1, bo1.reshape(1, D),
                     Wv2, bv2.reshape(1, D), Wo2, bo2.reshape(1, D))


# speedup vs baseline: 1.6738x; 1.6728x over previous
import functools

import jax
import jax.numpy as jnp
from jax import lax
from jax.experimental import pallas as pl
from jax.experimental.pallas import tpu as pltpu
from jax.experimental.pallas import tpu_sc as plsc

N = 10000
NP = 10240
E = 160000
B = 128
DIN = 1280
D = 256

NC = 2
NS = 16
LANES = 16

NPT = NP // (NC * NS)
DEG_CHUNK = 16000


def _rsqrt_newton(d):
    i = plsc.bitcast(d, jnp.int32)
    i = jnp.int32(0x5F3759DF) - (i >> 1)
    y = plsc.bitcast(i, jnp.float32)
    for _ in range(3):
        y = y * (1.5 - 0.5 * d * y * y)
    return y


def _deg_body(dst_hbm, dinv_hbm, dstv, deg, dinvv):
    c = lax.axis_index("c")
    s = lax.axis_index("s")
    base = (c * NS + s) * NPT
    zero16 = jnp.zeros((LANES,), jnp.float32)
    ones16 = jnp.ones((LANES,), jnp.float32)
    for i in range(NPT // LANES):
        deg[pl.ds(i * LANES, LANES)] = zero16

    for ch in range(E // DEG_CHUNK):
        pltpu.sync_copy(dst_hbm.at[pl.ds(ch * DEG_CHUNK, DEG_CHUNK)], dstv)

        def body(j, carry):
            v = dstv[pl.ds(j * LANES, LANES)]
            dloc = v - base
            m = (dloc >= 0) & (dloc < NPT)
            dloc = jnp.where(m, dloc, 0)
            plsc.addupdate_scatter(deg, [dloc], ones16, mask=m)
            return carry

        lax.fori_loop(0, DEG_CHUNK // LANES, body, 0)

    iota = lax.broadcasted_iota(jnp.int32, (LANES,), 0)
    for i in range(NPT // LANES):
        dg = deg[pl.ds(i * LANES, LANES)] + 1.0
        y = _rsqrt_newton(dg)
        nid = base + i * LANES + iota
        dinvv[pl.ds(i * LANES, LANES)] = jnp.where(nid < N, y, 0.0)
    pltpu.sync_copy(dinvv, dinv_hbm.at[pl.ds(base, NPT)])


def _sc_degree(dst):
    mesh = plsc.VectorSubcoreMesh(core_axis_name="c", subcore_axis_name="s",
                                  num_cores=NC, num_subcores=NS)
    f = pl.kernel(
        _deg_body,
        out_type=jax.ShapeDtypeStruct((NP,), jnp.float32),
        mesh=mesh,
        compiler_params=pltpu.CompilerParams(needs_layout_passes=False),
        scratch_types=[
            pltpu.VMEM((DEG_CHUNK,), jnp.int32),
            pltpu.VMEM((NPT,), jnp.float32),
            pltpu.VMEM((NPT,), jnp.float32),
        ],
    )
    return f(dst)


TPW = NP // (NC * NS)
TRASH_L = TPW
ACC_R = 336
KB = 32
CH = 2000
NCH = E // CH
STG = CH + KB + LANES


def _scatter_body(hws_hbm, src_hbm, dst_hbm, zer_hbm, x2_hbm,
                  srcv, dstv, cidx, cdloc, rows, acc, sem):
    c = lax.axis_index("c")
    s = lax.axis_index("s")
    lo = (c * NS + s) * TPW
    pltpu.sync_copy(zer_hbm, acc)

    trash16 = jnp.full((LANES,), TRASH_L, jnp.int32)
    pad16 = jnp.zeros((LANES,), jnp.int32)
    iota16 = lax.broadcasted_iota(jnp.int32, (LANES,), 0)

    def chunk_body(ch, carry):
        with jax.named_scope("edges_dma"):
            pltpu.sync_copy(src_hbm.at[pl.ds(ch * CH, CH)], srcv)
            pltpu.sync_copy(dst_hbm.at[pl.ds(ch * CH, CH)], dstv)

        def comp(j, cnt):
            sv = srcv[pl.ds(j * LANES, LANES)]
            dv = dstv[pl.ds(j * LANES, LANES)]
            dl = dv - lo
            m = (dl >= 0) & (dl < TPW)
            plsc.store_compressed(cidx.at[pl.ds(cnt, LANES)], sv, mask=m)
            plsc.store_compressed(cdloc.at[pl.ds(cnt, LANES)],
                                  jnp.where(m, dl, 0), mask=m)
            pc = plsc.all_reduce_population_count(m)
            return cnt + jnp.squeeze(lax.slice(pc, (0,), (1,)))

        with jax.named_scope("compact"):
            cnt = lax.fori_loop(0, CH // LANES, comp, jnp.int32(0))
            for j in range(KB // LANES):
                cidx[pl.ds(cnt + j * LANES, LANES)] = pad16
                cdloc[pl.ds(cnt + j * LANES, LANES)] = trash16

        def bat(b, carry2):
            with jax.named_scope("gather"):
                pltpu.async_copy(hws_hbm.at[cidx.at[pl.ds(b * KB, KB)]],
                                 rows, sem).wait()
            with jax.named_scope("accum"):
                for g in range(KB // LANES):
                    dl16 = cdloc[pl.ds(b * KB + g * LANES, LANES)]
                    e16 = iota16 + g * LANES

                    def _feat(j):
                        for jj in range(LANES):
                            fv = jnp.full((LANES,), 0, jnp.int32) + (j + jj)
                            v = plsc.load_gather(rows, [e16, fv])
                            plsc.addupdate_scatter(acc, [dl16, fv], v)

                    plsc.parallel_loop(0, D, LANES, unroll=2)(_feat)

            return carry2

        lax.fori_loop(0, (cnt + KB - 1) // KB, bat, 0)
        return carry

    lax.fori_loop(0, NCH, chunk_body, 0)
    pltpu.sync_copy(acc.at[pl.ds(0, TPW)], x2_hbm.at[pl.ds(lo, TPW)])


def _sc_scatter(hws, src, dst, zer):
    mesh = plsc.VectorSubcoreMesh(core_axis_name="c", subcore_axis_name="s",
                                  num_cores=NC, num_subcores=NS)
    f = pl.kernel(
        _scatter_body,
        out_type=jax.ShapeDtypeStruct((NP, D), jnp.float32),
        mesh=mesh,
        compiler_params=pltpu.CompilerParams(needs_layout_passes=False,
                                             disable_bounds_checks=True),
        scratch_types=[
            pltpu.VMEM((CH,), jnp.int32),
            pltpu.VMEM((CH,), jnp.int32),
            pltpu.VMEM((STG,), jnp.int32),
            pltpu.VMEM((STG,), jnp.int32),
            pltpu.VMEM((KB, D), jnp.float32),
            pltpu.VMEM((ACC_R, D), jnp.float32),
            pltpu.SemaphoreType.DMA,
        ],
    )
    return f(hws, src, dst, zer)


BM = 512


def _mm_body(x_ref, wt_ref, bt_ref, wg_ref, dinv_ref, hw_ref, hws_ref):
    h = jnp.dot(x_ref[...], wt_ref[...], preferred_element_type=jnp.float32)
    h = jnp.maximum(h + bt_ref[...], 0.0)
    hw = jnp.dot(h, wg_ref[...], preferred_element_type=jnp.float32)
    hw_ref[...] = hw
    hws_ref[...] = hw * dinv_ref[...]


def _tc_matmul(xp, Wt, bt2, Wg, dinv2):
    return pl.pallas_call(
        _mm_body,
        grid=(NP // BM,),
        in_specs=[
            pl.BlockSpec((BM, DIN), lambda i: (i, 0)),
            pl.BlockSpec((DIN, D), lambda i: (0, 0)),
            pl.BlockSpec((1, D), lambda i: (0, 0)),
            pl.BlockSpec((D, D), lambda i: (0, 0)),
            pl.BlockSpec((BM, 1), lambda i: (i, 0)),
        ],
        out_specs=[
            pl.BlockSpec((BM, D), lambda i: (i, 0)),
            pl.BlockSpec((BM, D), lambda i: (i, 0)),
        ],
        out_shape=[
            jax.ShapeDtypeStruct((NP, D), jnp.float32),
            jax.ShapeDtypeStruct((NP, D), jnp.float32),
        ],
    )(xp, Wt, bt2, Wg, dinv2)


BN = 1024


def _final_body(x2r_ref, hw_ref, dinv_ref, batch_ref, root_ref, bg_ref,
                wv1_ref, bv1_ref, wo1_ref, bo1_ref,
                wv2_ref, bv2_ref, wo2_ref, bo2_ref,
                out_ref, sacc, racc, cacc):
    i = pl.program_id(0)

    @pl.when(i == 0)
    def _init():
        sacc[...] = jnp.zeros_like(sacc)
        racc[...] = jnp.zeros_like(racc)
        cacc[...] = jnp.zeros_like(cacc)

    dv = dinv_ref[...]
    x2 = x2r_ref[...] * dv + hw_ref[...] * (dv * dv) + bg_ref[...]
    xr = jnp.maximum(x2, 0.0)

    bt = batch_ref[...]
    gid = lax.broadcasted_iota(jnp.int32, (B, BN), 0)
    sel = (bt == gid).astype(jnp.float32)
    sacc[...] += jnp.dot(sel, xr, preferred_element_type=jnp.float32)
    cacc[...] += jnp.sum(sel, axis=1, keepdims=True)

    nid = lax.broadcasted_iota(jnp.int32, (B, BN), 1) + i * BN
    rsel = (root_ref[...] == nid).astype(jnp.float32)
    racc[...] += jnp.dot(rsel, x2, preferred_element_type=jnp.float32)

    @pl.when(i == NP // BN - 1)
    def _fin():
        cnt = cacc[...]
        mean = sacc[...] / jnp.maximum(cnt, 1.0)
        o1 = jnp.dot(mean, wv1_ref[...], preferred_element_type=jnp.float32)
        o1 = jnp.dot(o1 + bv1_ref[...], wo1_ref[...],
                     preferred_element_type=jnp.float32) + bo1_ref[...]
        o2 = jnp.dot(racc[...], wv2_ref[...], preferred_element_type=jnp.float32)
        o2 = jnp.dot(o2 + bv2_ref[...], wo2_ref[...],
                     preferred_element_type=jnp.float32) + bo2_ref[...]
        out_ref[...] = jnp.where(cnt > 0.0,
                                 jnp.concatenate([o1, o2], axis=1), 0.0)


def _tc_final(x2raw, hw, dinv2, batchT, root2, bg2,
              Wv1, bv12, Wo1, bo12, Wv2, bv22, Wo2, bo22):
    full = lambda shape: pl.BlockSpec(shape, lambda i: (0, 0))
    return pl.pallas_call(
        _final_body,
        grid=(NP // BN,),
        in_specs=[
            pl.BlockSpec((BN, D), lambda i: (i, 0)),
            pl.BlockSpec((BN, D), lambda i: (i, 0)),
            pl.BlockSpec((BN, 1), lambda i: (i, 0)),
            pl.BlockSpec((1, BN), lambda i: (0, i)),
            full((B, 1)), full((1, D)),
            full((D, D)), full((1, D)), full((D, D)), full((1, D)),
            full((D, D)), full((1, D)), full((D, D)), full((1, D)),
        ],
        out_specs=pl.BlockSpec((B, 2 * D), lambda i: (0, 0)),
        out_shape=jax.ShapeDtypeStruct((B, 2 * D), jnp.float32),
        scratch_shapes=[
            pltpu.VMEM((B, D), jnp.float32),
            pltpu.VMEM((B, D), jnp.float32),
            pltpu.VMEM((B, 1), jnp.float32),
        ],
    )(x2raw, hw, dinv2, batchT, root2, bg2,
      Wv1, bv12, Wo1, bo12, Wv2, bv22, Wo2, bo22)


def kernel(x, edge_index, rootindex, batch, W_text, b_text, W_gcn, b_gcn,
           Wq1, bq1, Wk1, bk1, Wv1, bv1, Wq2, bq2, Wk2, bk2, Wv2, bv2,
           Wo1, bo1, Wo2, bo2):
    xp = jnp.pad(x, ((0, NP - N), (0, 0)))
    src = edge_index[0]
    dst = edge_index[1]

    dinv = _sc_degree(dst)
    dinv2 = dinv.reshape(NP, 1)
    hw, hws = _tc_matmul(xp, W_text, b_text.reshape(1, D), W_gcn, dinv2)

    zer = jnp.zeros((ACC_R, D), jnp.float32)
    x2raw = _sc_scatter(hws, src, dst, zer)

    batchT = jnp.pad(batch, (0, NP - N), constant_values=B).reshape(1, NP)
    return _tc_final(x2raw, hw, dinv2, batchT,
                     rootindex.reshape(B, 1), b_gcn.reshape(1, D),
                     Wv1, bv1.reshape(1, D), Wo1, bo1.reshape(1, D),
                     Wv2, bv2.reshape(1, D), Wo2, bo2.reshape(1, D))
